# Initial kernel scaffold; baseline (speedup 1.0000x reference)
import jax
import jax.numpy as jnp
from jax.experimental import pallas as pl


def kernel(z_e, embeddings):
    raise NotImplementedError("write your pallas kernel here")


# speedup vs baseline: 1.1581x; 1.1581x over previous
import functools

import jax
import jax.numpy as jnp
from jax import lax
from jax.experimental import pallas as pl
from jax.experimental.pallas import tpu as pltpu
from jax.experimental.pallas import tpu_sc as plsc

B = 16384
K = 8192
D = 64
CW = 0.25

BBLK = 256
KCH = 2048

NC = 2
NS = 16
NW = NC * NS
BPW = B // NW
ICH = 128
NCH = BPW // ICH


def _emb_norm_body(e_ref, o_ref):
    e = e_ref[...]
    nrm = jnp.sqrt(jnp.sum(e * e, axis=1, keepdims=True))
    o_ref[...] = e / jnp.maximum(nrm, 1e-12)


def _argmax_body(z_ref, emb_ref, idx_ref, mx_ref):
    z = z_ref[...]
    nrm = jnp.sqrt(jnp.sum(z * z, axis=1, keepdims=True))
    zn = z / jnp.maximum(nrm, 1e-12)

    def step(j, carry):
        mx, ix = carry
        eblk = emb_ref[pl.ds(j * KCH, KCH), :]
        sim = lax.dot_general(zn, eblk, (((1,), (1,)), ((), ())),
                              preferred_element_type=jnp.float32)
        cmx = jnp.max(sim, axis=1)
        iota = lax.broadcasted_iota(jnp.int32, sim.shape, 1)
        cix = jnp.min(jnp.where(sim == cmx[:, None], iota, KCH), axis=1)
        cix = cix + j * KCH
        better = cmx > mx
        return jnp.where(better, cmx, mx), jnp.where(better, cix, ix)

    mx0 = jnp.full((BBLK,), -jnp.inf, jnp.float32)
    ix0 = jnp.zeros((BBLK,), jnp.int32)
    mx, ix = lax.fori_loop(0, K // KCH, step, (mx0, ix0))
    idx_ref[0, 0, :] = ix
    mx_ref[0, 0, :] = mx


def _sc_gather_hist_body(emb_hbm, idx_hbm, zeros_hbm, ones_hbm,
                         zq_hbm, cnt_hbm,
                         idx_v, rows_v, ones_v, hist_sh, sem):
    c = lax.axis_index("c")
    s = lax.axis_index("s")
    wid = s * NC + c
    base = wid * BPW

    pltpu.sync_copy(idx_hbm.at[wid], idx_v)
    pltpu.sync_copy(ones_hbm, ones_v)

    @pl.when(s == 0)
    def _():
        pltpu.sync_copy(zeros_hbm, hist_sh)
    plsc.subcore_barrier()

    cps = [pltpu.async_copy(emb_hbm.at[idx_v.at[j]],
                            rows_v.at[pl.ds(j * ICH, ICH)], sem)
           for j in range(NCH)]
    for j in range(NCH):
        pltpu.sync_copy(ones_v, hist_sh.at[idx_v.at[j]], add=True)
    for cp in cps:
        cp.wait()
    pltpu.sync_copy(rows_v, zq_hbm.at[pl.ds(base, BPW)])

    plsc.subcore_barrier()

    @pl.when(s == 0)
    def _():
        pltpu.sync_copy(hist_sh, cnt_hbm.at[c])


def _scalars_body(mx_ref, cnt_ref, out_ref):
    mean_max = jnp.sum(mx_ref[...]) / B
    m = (2.0 - 2.0 * mean_max) / D
    cnt = jnp.sum(cnt_ref[...], axis=0)
    avg = cnt * (1.0 / B)
    ent = jnp.sum(avg * jnp.log(avg + 1e-10))
    out_ref[0] = CW * m
    out_ref[1] = m
    out_ref[2] = jnp.exp(-ent)
    out_ref[3] = jnp.sum((cnt > 0).astype(jnp.float32)) * (1.0 / K)


def kernel(z_e, embeddings):
    emb_norm = pl.pallas_call(
        _emb_norm_body,
        grid=(8,),
        in_specs=[pl.BlockSpec((K // 8, D), lambda i: (i, 0))],
        out_specs=pl.BlockSpec((K // 8, D), lambda i: (i, 0)),
        out_shape=jax.ShapeDtypeStruct((K, D), jnp.float32),
    )(embeddings)

    idx3, mx3 = pl.pallas_call(
        _argmax_body,
        grid=(B // BBLK,),
        in_specs=[
            pl.BlockSpec((BBLK, D), lambda i: (i, 0)),
            pl.BlockSpec((K, D), lambda i: (0, 0)),
        ],
        out_specs=[
            pl.BlockSpec((1, 1, BBLK), lambda i: (i, 0, 0)),
            pl.BlockSpec((1, 1, BBLK), lambda i: (i, 0, 0)),
        ],
        out_shape=[
            jax.ShapeDtypeStruct((B // BBLK, 1, BBLK), jnp.int32),
            jax.ShapeDtypeStruct((B // BBLK, 1, BBLK), jnp.float32),
        ],
    )(z_e, emb_norm)

    indices = idx3.reshape(B)

    sc_gather = functools.partial(
        pl.kernel,
        mesh=plsc.VectorSubcoreMesh(core_axis_name="c", subcore_axis_name="s"),
        out_type=[
            jax.ShapeDtypeStruct((B, D), jnp.float32),
            jax.ShapeDtypeStruct((NC, K), jnp.float32),
        ],
        scratch_types=[
            pltpu.VMEM((NCH, ICH), jnp.int32),
            pltpu.VMEM((BPW, D), jnp.float32),
            pltpu.VMEM((ICH,), jnp.float32),
            pltpu.VMEM_SHARED((K,), jnp.float32),
            pltpu.SemaphoreType.DMA,
        ],
        compiler_params=pltpu.CompilerParams(use_tc_tiling_on_sc=False),
    )(_sc_gather_hist_body)

    z_q, cnt2 = sc_gather(
        emb_norm,
        indices.reshape(NW, NCH, ICH),
        jnp.zeros((K,), jnp.float32),
        jnp.ones((ICH,), jnp.float32),
    )

    scal = pl.pallas_call(
        _scalars_body,
        out_specs=pl.BlockSpec(memory_space=pltpu.SMEM),
        out_shape=jax.ShapeDtypeStruct((4,), jnp.float32),
    )(mx3, cnt2)

    return (z_q, indices, scal[0], scal[1], scal[2], scal[3])


# speedup vs baseline: 2.1035x; 1.8163x over previous
import functools

import jax
import jax.numpy as jnp
from jax import lax
from jax.experimental import pallas as pl
from jax.experimental.pallas import tpu as pltpu
from jax.experimental.pallas import tpu_sc as plsc

B = 16384
K = 8192
D = 64
CW = 0.25
H = B // 2

BBLK = 512

NC = 2
NS = 16
NW = NC * NS
BPW = H // NW
ICH = 128
NCH = BPW // ICH


def _argmax_norm_body(z_ref, emb_ref, idx_ref, zn_ref, en_ref):
    @pl.when(pl.program_id(0) == 0)
    def _():
        e = emb_ref[...]
        enrm = jnp.sqrt(jnp.sum(e * e, axis=1, keepdims=True))
        en_ref[...] = e / jnp.maximum(enrm, 1e-12)

    z = z_ref[...]
    nrm = jnp.sqrt(jnp.sum(z * z, axis=1, keepdims=True))
    zn = z / jnp.maximum(nrm, 1e-12)
    zn_ref[...] = zn

    sim = lax.dot_general(zn, en_ref[...], (((1,), (1,)), ((), ())),
                          preferred_element_type=jnp.float32)
    idx_ref[0, 0, :] = jnp.argmax(sim, axis=1).astype(jnp.int32)


def _argmax_body(z_ref, en_ref, idx_ref, zn_ref):
    z = z_ref[...]
    nrm = jnp.sqrt(jnp.sum(z * z, axis=1, keepdims=True))
    zn = z / jnp.maximum(nrm, 1e-12)
    zn_ref[...] = zn

    sim = lax.dot_general(zn, en_ref[...], (((1,), (1,)), ((), ())),
                          preferred_element_type=jnp.float32)
    idx_ref[0, 0, :] = jnp.argmax(sim, axis=1).astype(jnp.int32)


def _sc_gather_hist_body(emb_hbm, idx_hbm, zeros_hbm, ones_hbm,
                         zq_hbm, cnt_hbm,
                         idx_v, rows_v, ones_v, hist_sh, sem):
    c = lax.axis_index("c")
    s = lax.axis_index("s")
    wid = s * NC + c
    base = wid * BPW

    pltpu.sync_copy(idx_hbm.at[wid], idx_v)
    pltpu.sync_copy(ones_hbm, ones_v)

    @pl.when(s == 0)
    def _():
        pltpu.sync_copy(zeros_hbm, hist_sh)
    plsc.subcore_barrier()

    cps = [pltpu.async_copy(emb_hbm.at[idx_v.at[j]],
                            rows_v.at[pl.ds(j * ICH, ICH)], sem)
           for j in range(NCH)]
    for j in range(NCH):
        pltpu.sync_copy(ones_v.at[j], hist_sh.at[idx_v.at[j]], add=True)
    for cp in cps:
        cp.wait()
    pltpu.sync_copy(rows_v, zq_hbm.at[pl.ds(base, BPW)])

    plsc.subcore_barrier()

    @pl.when(s == 0)
    def _():
        pltpu.sync_copy(hist_sh, cnt_hbm.at[c])


def _scalars_body(zn1_ref, zn2_ref, zq1_ref, zq2_ref, cnt_ref, out_ref):
    d1 = zn1_ref[...] - zq1_ref[...]
    d2 = zn2_ref[...] - zq2_ref[...]
    m = (jnp.sum(d1 * d1) + jnp.sum(d2 * d2)) / (B * D)
    cnt = jnp.sum(cnt_ref[...], axis=0)
    avg = cnt * (1.0 / B)
    ent = jnp.sum(avg * jnp.log(avg + 1e-10))
    out_ref[0] = CW * m
    out_ref[1] = m
    out_ref[2] = jnp.exp(-ent)
    out_ref[3] = jnp.sum((cnt > 0).astype(jnp.float32)) * (1.0 / K)


def _make_sc_gather():
    return functools.partial(
        pl.kernel,
        mesh=plsc.VectorSubcoreMesh(core_axis_name="c", subcore_axis_name="s"),
        out_type=[
            jax.ShapeDtypeStruct((H, D), jnp.float32),
            jax.ShapeDtypeStruct((NC, K), jnp.float32),
        ],
        scratch_types=[
            pltpu.VMEM((NCH, ICH), jnp.int32),
            pltpu.VMEM((BPW, D), jnp.float32),
            pltpu.VMEM((NCH, ICH), jnp.float32),
            pltpu.VMEM_SHARED((K,), jnp.float32),
            pltpu.SemaphoreType.DMA,
        ],
        compiler_params=pltpu.CompilerParams(use_tc_tiling_on_sc=False),
    )(_sc_gather_hist_body)


def kernel(z_e, embeddings):
    idx3_1, zn1, emb_norm = pl.pallas_call(
        _argmax_norm_body,
        grid=(H // BBLK,),
        in_specs=[
            pl.BlockSpec((BBLK, D), lambda i: (i, 0)),
            pl.BlockSpec((K, D), lambda i: (0, 0)),
        ],
        out_specs=[
            pl.BlockSpec((1, 1, BBLK), lambda i: (i, 0, 0)),
            pl.BlockSpec((BBLK, D), lambda i: (i, 0)),
            pl.BlockSpec((K, D), lambda i: (0, 0)),
        ],
        out_shape=[
            jax.ShapeDtypeStruct((H // BBLK, 1, BBLK), jnp.int32),
            jax.ShapeDtypeStruct((H, D), jnp.float32),
            jax.ShapeDtypeStruct((K, D), jnp.float32),
        ],
    )(z_e[:H], embeddings)

    sc_gather = _make_sc_gather()
    zeros = jnp.zeros((K,), jnp.float32)
    ones = jnp.ones((NCH, ICH), jnp.float32)

    idx1 = idx3_1.reshape(H)
    zq1, cnt2_1 = sc_gather(emb_norm, idx1.reshape(NW, NCH, ICH), zeros, ones)

    idx3_2, zn2 = pl.pallas_call(
        _argmax_body,
        grid=(H // BBLK,),
        in_specs=[
            pl.BlockSpec((BBLK, D), lambda i: (i, 0)),
            pl.BlockSpec((K, D), lambda i: (0, 0)),
        ],
        out_specs=[
            pl.BlockSpec((1, 1, BBLK), lambda i: (i, 0, 0)),
            pl.BlockSpec((BBLK, D), lambda i: (i, 0)),
        ],
        out_shape=[
            jax.ShapeDtypeStruct((H // BBLK, 1, BBLK), jnp.int32),
            jax.ShapeDtypeStruct((H, D), jnp.float32),
        ],
    )(z_e[H:], emb_norm)

    idx2 = idx3_2.reshape(H)
    zq2, cnt2_2 = sc_gather(emb_norm, idx2.reshape(NW, NCH, ICH), zeros, ones)

    cnt4 = jnp.concatenate([cnt2_1, cnt2_2], axis=0)

    scal = pl.pallas_call(
        _scalars_body,
        out_specs=pl.BlockSpec(memory_space=pltpu.SMEM),
        out_shape=jax.ShapeDtypeStruct((4,), jnp.float32),
    )(zn1, zn2, zq1, zq2, cnt4)

    z_q = jnp.concatenate([zq1, zq2], axis=0)
    indices = jnp.concatenate([idx1, idx2], axis=0)
    return (z_q, indices, scal[0], scal[1], scal[2], scal[3])


# speedup vs baseline: 2.3606x; 1.1222x over previous
import functools

import jax
import jax.numpy as jnp
from jax import lax
from jax.experimental import pallas as pl
from jax.experimental.pallas import tpu as pltpu
from jax.experimental.pallas import tpu_sc as plsc

B = 16384
K = 8192
D = 64
CW = 0.25

BBLK = 512

NC = 2
NS = 16
NW = NC * NS
BPW = B // NW
ICH = 128
NCH = BPW // ICH


def _argmax_body(z_ref, emb_ref, idx_ref, idxsc_ref, zn_ref, en_ref):
    @pl.when(pl.program_id(0) == 0)
    def _():
        e = emb_ref[...]
        enrm = jnp.sqrt(jnp.sum(e * e, axis=1, keepdims=True))
        en_ref[...] = e / jnp.maximum(enrm, 1e-12)

    z = z_ref[...]
    nrm = jnp.sqrt(jnp.sum(z * z, axis=1, keepdims=True))
    zn = z / jnp.maximum(nrm, 1e-12)
    zn_ref[...] = zn

    sim = lax.dot_general(zn, en_ref[...], (((1,), (1,)), ((), ())),
                          preferred_element_type=jnp.float32)
    ix = jnp.argmax(sim, axis=1).astype(jnp.int32)
    idx_ref[...] = ix
    idxsc_ref[0] = ix.reshape(BBLK // ICH, ICH)


def _sc_gather_hist_body(emb_hbm, idx_hbm, zeros_hbm, ones_hbm,
                         zq_hbm, cnt_hbm,
                         idx_v, rows_v, ones_v, hist_sh, sem):
    c = lax.axis_index("c")
    s = lax.axis_index("s")
    wid = s * NC + c
    base = wid * BPW

    pltpu.sync_copy(idx_hbm.at[wid], idx_v)
    pltpu.sync_copy(ones_hbm, ones_v)

    @pl.when(s == 0)
    def _():
        pltpu.sync_copy(zeros_hbm, hist_sh)
    plsc.subcore_barrier()

    cps = [pltpu.async_copy(emb_hbm.at[idx_v.at[j]],
                            rows_v.at[pl.ds(j * ICH, ICH)], sem)
           for j in range(NCH)]
    for j in range(NCH):
        pltpu.sync_copy(ones_v.at[j], hist_sh.at[idx_v.at[j]], add=True)
    for cp in cps:
        cp.wait()
    pltpu.sync_copy(rows_v, zq_hbm.at[pl.ds(base, BPW)])

    plsc.subcore_barrier()

    @pl.when(s == 0)
    def _():
        pltpu.sync_copy(hist_sh, cnt_hbm.at[c])


def _scalars_body(zn_ref, zq_ref, cnt_ref, out_ref):
    diff = zn_ref[...] - zq_ref[...]
    m = jnp.sum(diff * diff) / (B * D)
    cnt = jnp.sum(cnt_ref[...], axis=0)
    avg = cnt * (1.0 / B)
    ent = jnp.sum(avg * jnp.log(avg + 1e-10))
    out_ref[0] = CW * m
    out_ref[1] = m
    out_ref[2] = jnp.exp(-ent)
    out_ref[3] = jnp.sum((cnt > 0).astype(jnp.float32)) * (1.0 / K)


def kernel(z_e, embeddings):
    indices, idxsc, zn, emb_norm = pl.pallas_call(
        _argmax_body,
        grid=(B // BBLK,),
        in_specs=[
            pl.BlockSpec((BBLK, D), lambda i: (i, 0)),
            pl.BlockSpec((K, D), lambda i: (0, 0)),
        ],
        out_specs=[
            pl.BlockSpec((BBLK,), lambda i: (i,)),
            pl.BlockSpec((1, BBLK // ICH, ICH), lambda i: (i, 0, 0)),
            pl.BlockSpec((BBLK, D), lambda i: (i, 0)),
            pl.BlockSpec((K, D), lambda i: (0, 0)),
        ],
        out_shape=[
            jax.ShapeDtypeStruct((B,), jnp.int32),
            jax.ShapeDtypeStruct((NW, NCH, ICH), jnp.int32),
            jax.ShapeDtypeStruct((B, D), jnp.float32),
            jax.ShapeDtypeStruct((K, D), jnp.float32),
        ],
    )(z_e, embeddings)

    sc_gather = functools.partial(
        pl.kernel,
        mesh=plsc.VectorSubcoreMesh(core_axis_name="c", subcore_axis_name="s"),
        out_type=[
            jax.ShapeDtypeStruct((B, D), jnp.float32),
            jax.ShapeDtypeStruct((NC, K), jnp.float32),
        ],
        scratch_types=[
            pltpu.VMEM((NCH, ICH), jnp.int32),
            pltpu.VMEM((BPW, D), jnp.float32),
            pltpu.VMEM((NCH, ICH), jnp.float32),
            pltpu.VMEM_SHARED((K,), jnp.float32),
            pltpu.SemaphoreType.DMA,
        ],
        compiler_params=pltpu.CompilerParams(use_tc_tiling_on_sc=False),
    )(_sc_gather_hist_body)

    z_q, cnt2 = sc_gather(
        emb_norm,
        idxsc,
        jnp.zeros((K,), jnp.float32),
        jnp.ones((NCH, ICH), jnp.float32),
    )

    scal = pl.pallas_call(
        _scalars_body,
        out_specs=pl.BlockSpec(memory_space=pltpu.SMEM),
        out_shape=jax.ShapeDtypeStruct((4,), jnp.float32),
    )(zn, z_q, cnt2)

    return (z_q, indices, scal[0], scal[1], scal[2], scal[3])


# speedup vs baseline: 2.3911x; 1.0129x over previous
import functools

import jax
import jax.numpy as jnp
from jax import lax
from jax.experimental import pallas as pl
from jax.experimental.pallas import tpu as pltpu
from jax.experimental.pallas import tpu_sc as plsc

B = 16384
K = 8192
D = 64
CW = 0.25

BBLK = 512

NC = 2
NS = 16
NW = NC * NS
BPW = B // NW
ICH = 128
NCH = BPW // ICH


def _argmax_body(z_ref, emb_ref, idx_ref, idxsc_ref, zn_ref, en_ref):
    @pl.when(pl.program_id(0) == 0)
    def _():
        e = emb_ref[...]
        enrm = jnp.sqrt(jnp.sum(e * e, axis=1, keepdims=True))
        en_ref[...] = e / jnp.maximum(enrm, 1e-12)

    z = z_ref[...]
    nrm = jnp.sqrt(jnp.sum(z * z, axis=1, keepdims=True))
    zn = z / jnp.maximum(nrm, 1e-12)
    zn_ref[...] = zn

    sim = lax.dot_general(zn, en_ref[...], (((1,), (1,)), ((), ())),
                          preferred_element_type=jnp.float32)
    ix = jnp.argmax(sim, axis=1).astype(jnp.int32)
    idx_ref[...] = ix
    idxsc_ref[0] = ix.reshape(BBLK // ICH, ICH)


def _sc_gather_hist_body(emb_hbm, idx_hbm,
                         zq_hbm, cnt_hbm,
                         idx_v, rows_v, ones_v, zed_v, hist_sh, sem):
    c = lax.axis_index("c")
    s = lax.axis_index("s")
    wid = s * NC + c
    base = wid * BPW
    zslice = K // NS

    pltpu.sync_copy(idx_hbm.at[wid], idx_v)
    one16 = jnp.ones((16,), jnp.float32)
    zero16 = jnp.zeros((16,), jnp.float32)
    for j in range(NCH):
        for t in range(ICH // 16):
            ones_v[j, pl.ds(t * 16, 16)] = one16
    for t in range(zslice // 16):
        zed_v[pl.ds(t * 16, 16)] = zero16
    pltpu.sync_copy(zed_v, hist_sh.at[pl.ds(s * zslice, zslice)])
    plsc.subcore_barrier()

    cps = [pltpu.async_copy(emb_hbm.at[idx_v.at[j]],
                            rows_v.at[pl.ds(j * ICH, ICH)], sem)
           for j in range(NCH)]
    for j in range(NCH):
        pltpu.sync_copy(ones_v.at[j], hist_sh.at[idx_v.at[j]], add=True)
    for cp in cps:
        cp.wait()
    pltpu.sync_copy(rows_v, zq_hbm.at[pl.ds(base, BPW)])

    plsc.subcore_barrier()

    @pl.when(s == 0)
    def _():
        pltpu.sync_copy(hist_sh, cnt_hbm.at[c])


SGRID = 8


def _scalars_body(zn_ref, zq_ref, cnt_ref, out_ref, acc_ref):
    i = pl.program_id(0)
    diff = zn_ref[...] - zq_ref[...]
    ps = jnp.sum(diff * diff)

    @pl.when(i == 0)
    def _():
        acc_ref[0] = ps

    @pl.when(i > 0)
    def _():
        acc_ref[0] = acc_ref[0] + ps

    @pl.when(i == SGRID - 1)
    def _():
        m = acc_ref[0] / (B * D)
        cnt = jnp.sum(cnt_ref[...], axis=0)
        avg = cnt * (1.0 / B)
        ent = jnp.sum(avg * jnp.log(avg + 1e-10))
        out_ref[0] = CW * m
        out_ref[1] = m
        out_ref[2] = jnp.exp(-ent)
        out_ref[3] = jnp.sum((cnt > 0).astype(jnp.float32)) * (1.0 / K)


def kernel(z_e, embeddings):
    indices, idxsc, zn, emb_norm = pl.pallas_call(
        _argmax_body,
        grid=(B // BBLK,),
        in_specs=[
            pl.BlockSpec((BBLK, D), lambda i: (i, 0)),
            pl.BlockSpec((K, D), lambda i: (0, 0)),
        ],
        out_specs=[
            pl.BlockSpec((BBLK,), lambda i: (i,)),
            pl.BlockSpec((1, BBLK // ICH, ICH), lambda i: (i, 0, 0)),
            pl.BlockSpec((BBLK, D), lambda i: (i, 0)),
            pl.BlockSpec((K, D), lambda i: (0, 0)),
        ],
        out_shape=[
            jax.ShapeDtypeStruct((B,), jnp.int32),
            jax.ShapeDtypeStruct((NW, NCH, ICH), jnp.int32),
            jax.ShapeDtypeStruct((B, D), jnp.float32),
            jax.ShapeDtypeStruct((K, D), jnp.float32),
        ],
    )(z_e, embeddings)

    sc_gather = functools.partial(
        pl.kernel,
        mesh=plsc.VectorSubcoreMesh(core_axis_name="c", subcore_axis_name="s"),
        out_type=[
            jax.ShapeDtypeStruct((B, D), jnp.float32),
            jax.ShapeDtypeStruct((NC, K), jnp.float32),
        ],
        scratch_types=[
            pltpu.VMEM((NCH, ICH), jnp.int32),
            pltpu.VMEM((BPW, D), jnp.float32),
            pltpu.VMEM((NCH, ICH), jnp.float32),
            pltpu.VMEM((K // NS,), jnp.float32),
            pltpu.VMEM_SHARED((K,), jnp.float32),
            pltpu.SemaphoreType.DMA,
        ],
        compiler_params=pltpu.CompilerParams(use_tc_tiling_on_sc=False),
    )(_sc_gather_hist_body)

    z_q, cnt2 = sc_gather(emb_norm, idxsc)

    scal = pl.pallas_call(
        _scalars_body,
        grid=(SGRID,),
        in_specs=[
            pl.BlockSpec((B // SGRID, D), lambda i: (i, 0)),
            pl.BlockSpec((B // SGRID, D), lambda i: (i, 0)),
            pl.BlockSpec((NC, K), lambda i: (0, 0)),
        ],
        out_specs=pl.BlockSpec(memory_space=pltpu.SMEM),
        out_shape=jax.ShapeDtypeStruct((4,), jnp.float32),
        scratch_shapes=[pltpu.SMEM((1,), jnp.float32)],
    )(zn, z_q, cnt2)

    return (z_q, indices, scal[0], scal[1], scal[2], scal[3])
